# baseline (speedup 1.0000x reference)
import functools

import jax
import jax.numpy as jnp
from jax import lax
from jax.experimental import pallas as pl
from jax.experimental.pallas import tpu as pltpu
from jax.experimental.pallas import tpu_sc as plsc

_B, _L, _D = 4, 2048, 1024
_NC, _NS = 2, 16
_NW = _NC * _NS
_RW = _L // _NW
_CH = 16
_NCH = _RW // _CH
_LANES = 16


def _sc_body(x_hbm, pe_hbm, out_hbm, pe_v, xbuf, sem):
    wid = lax.axis_index("s") * _NC + lax.axis_index("c")
    base = wid * _RW
    pltpu.sync_copy(pe_hbm.at[pl.ds(base, _RW)], pe_v)

    def batch_body(b, _):
        def chunk_body(c, _):
            buf = xbuf.at[c % 2]
            pltpu.async_copy(
                x_hbm.at[b, pl.ds(base + c * _CH, _CH)], buf, sem
            ).wait()

            def row_body(r, _):
                def vec_body(k, _):
                    off = k * _LANES
                    buf[r, pl.ds(off, _LANES)] = (
                        buf[r, pl.ds(off, _LANES)]
                        + pe_v[c * _CH + r, pl.ds(off, _LANES)]
                    )
                    return 0
                return lax.fori_loop(0, _D // _LANES, vec_body, 0)

            lax.fori_loop(0, _CH, row_body, 0)
            pltpu.sync_copy(buf, out_hbm.at[b, pl.ds(base + c * _CH, _CH)])
            return 0

        return lax.fori_loop(0, _NCH, chunk_body, 0)

    lax.fori_loop(0, _B, batch_body, 0)


def kernel(x, pe):
    mesh = plsc.VectorSubcoreMesh(core_axis_name="c", subcore_axis_name="s")
    sc_add = functools.partial(
        pl.kernel,
        mesh=mesh,
        out_type=jax.ShapeDtypeStruct((_B, _L, _D), jnp.float32),
        scratch_types=[
            pltpu.VMEM((_RW, _D), jnp.float32),
            pltpu.VMEM((2, _CH, _D), jnp.float32),
            pltpu.SemaphoreType.DMA,
        ],
    )(_sc_body)
    return sc_add(x, pe)


# speedup vs baseline: 1.0679x; 1.0679x over previous
import functools

import jax
import jax.numpy as jnp
from jax import lax
from jax.experimental import pallas as pl
from jax.experimental.pallas import tpu as pltpu
from jax.experimental.pallas import tpu_sc as plsc

_B, _L, _D = 4, 2048, 1024
_NC, _NS = 2, 16
_NW = _NC * _NS
_RW = _L // _NW
_CH = 16
_NCH = _RW // _CH
_CW = _CH * _D
_NBUF = 3
_LANES = 16
_PAIRS = [(b, c) for b in range(_B) for c in range(_NCH)]


def _sc_body(x_hbm, pe_hbm, out_hbm, pe_v, *scratch):
    xbuf = scratch[:_NBUF]
    s_pe = scratch[_NBUF]
    sems = scratch[_NBUF + 1:]
    s_in, s_out = sems[:_NBUF], sems[_NBUF:]
    wid = lax.axis_index("s") * _NC + lax.axis_index("c")
    base = wid * (_RW * _D)

    def in_copy(i):
        b, c = _PAIRS[i]
        return pltpu.make_async_copy(
            x_hbm.at[b, pl.ds(base + c * _CW, _CW)],
            xbuf[i % _NBUF],
            s_in[i % _NBUF],
        )

    def out_copy(i):
        b, c = _PAIRS[i]
        return pltpu.make_async_copy(
            xbuf[i % _NBUF],
            out_hbm.at[b, pl.ds(base + c * _CW, _CW)],
            s_out[i % _NBUF],
        )

    pltpu.make_async_copy(
        pe_hbm.at[pl.ds(base, _RW * _D)], pe_v, s_pe
    ).start()
    for i in range(_NBUF):
        in_copy(i).start()
    pltpu.make_async_copy(
        pe_hbm.at[pl.ds(base, _RW * _D)], pe_v, s_pe
    ).wait()

    n = len(_PAIRS)
    for i in range(n):
        if 0 <= i - 1 and i + _NBUF - 1 < n:
            out_copy(i - 1).wait()
            in_copy(i + _NBUF - 1).start()
        in_copy(i).wait()
        buf = xbuf[i % _NBUF]
        pe_off = _PAIRS[i][1] * _CW

        def vec_body(k, _, pe_off=pe_off, buf=buf):
            o = k * _LANES
            buf[pl.ds(o, _LANES)] = (
                buf[pl.ds(o, _LANES)] + pe_v[pl.ds(pe_off + o, _LANES)]
            )
            return 0

        lax.fori_loop(0, _CW // _LANES, vec_body, 0, unroll=8)
        out_copy(i).start()
    for i in range(n - _NBUF + 1, n):
        out_copy(i).wait()


def kernel(x, pe):
    mesh = plsc.VectorSubcoreMesh(core_axis_name="c", subcore_axis_name="s")
    sc_add = functools.partial(
        pl.kernel,
        mesh=mesh,
        out_type=jax.ShapeDtypeStruct((_B, _L * _D), jnp.float32),
        scratch_types=(
            [
                pltpu.VMEM((_RW * _D,), jnp.float32),
            ]
            + [pltpu.VMEM((_CW,), jnp.float32)] * _NBUF
            + [pltpu.SemaphoreType.DMA] * (1 + 2 * _NBUF)
        ),
    )(_sc_body)
    out = sc_add(x.reshape(_B, _L * _D), pe.reshape(_L * _D))
    return out.reshape(_B, _L, _D)


# speedup vs baseline: 1.2049x; 1.1282x over previous
import functools

import jax
import jax.numpy as jnp
from jax import lax
from jax.experimental import pallas as pl
from jax.experimental.pallas import tpu as pltpu
from jax.experimental.pallas import tpu_sc as plsc

_B, _L, _D = 4, 2048, 1024
_NC, _NS = 2, 16
_NW = _NC * _NS
_RW = _L // _NW
_CH = 16
_NCH = _RW // _CH
_CW = _CH * _D
_NBUF = 3
_LANES = 16
_PAIRS = [(b, c) for b in range(_B) for c in range(_NCH)]


def _sc_body(x_hbm, pe_hbm, out_hbm, pe_v, *scratch):
    xbuf = scratch[:_NBUF]
    s_pe = scratch[_NBUF]
    sems = scratch[_NBUF + 1:]
    s_in, s_out = sems[:_NBUF], sems[_NBUF:]
    wid = lax.axis_index("s") * _NC + lax.axis_index("c")
    base = wid * (_RW * _D)

    def in_copy(i):
        b, c = _PAIRS[i]
        return pltpu.make_async_copy(
            x_hbm.at[b, pl.ds(base + c * _CW, _CW)],
            xbuf[i % _NBUF],
            s_in[i % _NBUF],
        )

    def out_copy(i):
        b, c = _PAIRS[i]
        return pltpu.make_async_copy(
            xbuf[i % _NBUF],
            out_hbm.at[b, pl.ds(base + c * _CW, _CW)],
            s_out[i % _NBUF],
        )

    pltpu.make_async_copy(
        pe_hbm.at[pl.ds(base, _RW * _D)], pe_v, s_pe
    ).start()
    for i in range(_NBUF):
        in_copy(i).start()
    pltpu.make_async_copy(
        pe_hbm.at[pl.ds(base, _RW * _D)], pe_v, s_pe
    ).wait()

    n = len(_PAIRS)
    for i in range(n):
        if 0 <= i - 1 and i + _NBUF - 1 < n:
            out_copy(i - 1).wait()
            in_copy(i + _NBUF - 1).start()
        in_copy(i).wait()
        buf = xbuf[i % _NBUF]
        pe_off = _PAIRS[i][1] * _CW

        @plsc.parallel_loop(0, _CW, step=_LANES, unroll=8)
        def vec_body(o, pe_off=pe_off, buf=buf):
            buf[pl.ds(o, _LANES)] = (
                buf[pl.ds(o, _LANES)] + pe_v[pl.ds(pe_off + o, _LANES)]
            )

        out_copy(i).start()
    for i in range(n - _NBUF + 1, n):
        out_copy(i).wait()


def kernel(x, pe):
    mesh = plsc.VectorSubcoreMesh(core_axis_name="c", subcore_axis_name="s")
    sc_add = functools.partial(
        pl.kernel,
        mesh=mesh,
        out_type=jax.ShapeDtypeStruct((_B, _L * _D), jnp.float32),
        scratch_types=(
            [
                pltpu.VMEM((_RW * _D,), jnp.float32),
            ]
            + [pltpu.VMEM((_CW,), jnp.float32)] * _NBUF
            + [pltpu.SemaphoreType.DMA] * (1 + 2 * _NBUF)
        ),
    )(_sc_body)
    out = sc_add(x.reshape(_B, _L * _D), pe.reshape(_L * _D))
    return out.reshape(_B, _L, _D)


# speedup vs baseline: 6.0332x; 5.0072x over previous
import jax
import jax.numpy as jnp
from jax.experimental import pallas as pl


_BL = 2048


def _body(x_ref, pe_ref, o_ref):
    o_ref[...] = x_ref[...] + pe_ref[...]


def kernel(x, pe):
    b, l, d = x.shape
    nl = l // _BL
    return pl.pallas_call(
        _body,
        grid=(nl, b),
        in_specs=[
            pl.BlockSpec((1, _BL, d), lambda i, j: (j, i, 0)),
            pl.BlockSpec((_BL, d), lambda i, j: (i, 0)),
        ],
        out_specs=pl.BlockSpec((1, _BL, d), lambda i, j: (j, i, 0)),
        out_shape=jax.ShapeDtypeStruct((b, l, d), x.dtype),
    )(x, pe)
